# baseline (speedup 1.0000x reference)
import jax
import jax.numpy as jnp
from jax.experimental import pallas as pl
from jax.experimental.pallas import tpu as pltpu

FEATURES = 256
NUM_HEADS = 8
CODEBOOK_SIZE = 2048
HEAD_FEATURES = FEATURES // NUM_HEADS
EXPIRE_THRESHOLD = 2
B, N = 4, 1024
BN = B * N
NT = 512
N_TILES = BN // NT


def _vqe_kernel(x_ref, c_ref, ema_ref,
                out_ref, idx_ref, loss_ref, perp_ref, repl_ref,
                counts_ref):
    nt = pl.program_id(0)

    loss_partial = None
    for h in range(NUM_HEADS):
        q = x_ref[h]
        c = c_ref[h]

        dot = jnp.dot(q, c.T, preferred_element_type=jnp.float32)
        l2_q = jnp.sum(q * q, axis=1, keepdims=True)
        l2_c = jnp.sum(c * c, axis=1)[None, :]
        sim = -((l2_q + l2_c) - 2.0 * dot)
        idx = jnp.argmax(sim, axis=1).astype(jnp.int32)
        idx_ref[h, :] = idx

        iota_m = jax.lax.broadcasted_iota(jnp.int32, (NT, CODEBOOK_SIZE), 1)
        tile_counts = jnp.sum((idx[:, None] == iota_m).astype(jnp.float32),
                              axis=0)

        @pl.when(nt == 0)
        def _init_counts():
            counts_ref[h, :] = tile_counts

        @pl.when(nt != 0)
        def _acc_counts():
            counts_ref[h, :] += tile_counts

        s = jnp.sum(c, axis=0)
        out_ref[h] = jnp.broadcast_to(s[None, :], (NT, HEAD_FEATURES))

        part = jnp.sum((q - s[None, :]) ** 2) * (1.0 / (B * N * FEATURES))
        loss_partial = part if loss_partial is None else loss_partial + part

    @pl.when(nt == 0)
    def _init_loss():
        loss_ref[0, :] = jnp.zeros((128,), jnp.float32)

    loss_ref[0, :] += jnp.full((128,), loss_partial, jnp.float32)

    @pl.when(nt == 0)
    def _repl():
        n_exp = jnp.sum((ema_ref[...] < float(EXPIRE_THRESHOLD))
                        .astype(jnp.float32), axis=1)
        repl_ref[...] = jnp.broadcast_to(n_exp[:, None], (NUM_HEADS, 128))

    @pl.when(nt == N_TILES - 1)
    def _perp():
        mean = counts_ref[...] * (1.0 / BN)
        ent = -jnp.sum(mean * jnp.log(mean + 1e-10), axis=1)
        perp_ref[...] = jnp.broadcast_to(jnp.exp(ent)[:, None],
                                         (NUM_HEADS, 128))


@jax.jit
def kernel(x, codebooks, ema_cluster_size):
    xh = x.reshape(B * N, NUM_HEADS, HEAD_FEATURES).transpose(1, 0, 2)

    grid = (N_TILES,)
    out_h, idx_out, loss_out, perp_out, repl_out = pl.pallas_call(
        _vqe_kernel,
        grid=grid,
        in_specs=[
            pl.BlockSpec((NUM_HEADS, NT, HEAD_FEATURES), lambda nt: (0, nt, 0)),
            pl.BlockSpec((NUM_HEADS, CODEBOOK_SIZE, HEAD_FEATURES),
                         lambda nt: (0, 0, 0)),
            pl.BlockSpec((NUM_HEADS, CODEBOOK_SIZE), lambda nt: (0, 0)),
        ],
        out_specs=[
            pl.BlockSpec((NUM_HEADS, NT, HEAD_FEATURES), lambda nt: (0, nt, 0)),
            pl.BlockSpec((NUM_HEADS, NT), lambda nt: (0, nt)),
            pl.BlockSpec((1, 128), lambda nt: (0, 0)),
            pl.BlockSpec((NUM_HEADS, 128), lambda nt: (0, 0)),
            pl.BlockSpec((NUM_HEADS, 128), lambda nt: (0, 0)),
        ],
        out_shape=[
            jax.ShapeDtypeStruct((NUM_HEADS, BN, HEAD_FEATURES), jnp.float32),
            jax.ShapeDtypeStruct((NUM_HEADS, BN), jnp.int32),
            jax.ShapeDtypeStruct((1, 128), jnp.float32),
            jax.ShapeDtypeStruct((NUM_HEADS, 128), jnp.float32),
            jax.ShapeDtypeStruct((NUM_HEADS, 128), jnp.float32),
        ],
        scratch_shapes=[pltpu.VMEM((NUM_HEADS, CODEBOOK_SIZE), jnp.float32)],
    )(xh, codebooks, ema_cluster_size)

    out = out_h.transpose(1, 0, 2).reshape(B, N, FEATURES)
    codebook_indices = idx_out.reshape(NUM_HEADS, B, N).transpose(1, 0, 2)
    loss = loss_out[0, 0]
    perp = perp_out[:, 0]
    replaced_codes = repl_out[:, 0].astype(jnp.int32)
    return out, codebook_indices, loss, perp, replaced_codes
